# Initial kernel scaffold; baseline (speedup 1.0000x reference)
import jax
import jax.numpy as jnp
from jax.experimental import pallas as pl


def kernel(table, indices):
    raise NotImplementedError("write your pallas kernel here")


# speedup vs baseline: 1.8322x; 1.8322x over previous
import functools

import jax
import jax.numpy as jnp
from jax import lax
from jax.experimental import pallas as pl
from jax.experimental.pallas import tpu as pltpu
from jax.experimental.pallas import tpu_sc as plsc

_D = 64
_N = 16384 * 50
_NC = 2
_NS = 16
_NW = _NC * _NS
_PER_W = _N // _NW
_CHUNK = 800
_NCHUNK = _PER_W // _CHUNK


def _make_gather():
    mesh = plsc.VectorSubcoreMesh(core_axis_name="c", subcore_axis_name="s")

    @functools.partial(
        pl.kernel,
        mesh=mesh,
        out_type=jax.ShapeDtypeStruct((_N, _D), jnp.float32),
        scratch_types=[
            pltpu.VMEM((_CHUNK,), jnp.int32),
            pltpu.VMEM((_CHUNK, _D), jnp.float32),
            pltpu.SemaphoreType.DMA,
        ],
        compiler_params=pltpu.CompilerParams(use_tc_tiling_on_sc=False),
    )
    def gather(table_hbm, idx_hbm, out_hbm, idx_v, rows_v, sem):
        wid = lax.axis_index("s") * _NC + lax.axis_index("c")
        base = wid * _PER_W

        def body(i, carry):
            off = base + i * _CHUNK
            pltpu.sync_copy(idx_hbm.at[pl.ds(off, _CHUNK)], idx_v)
            pltpu.async_copy(table_hbm.at[idx_v], rows_v, sem).wait()
            pltpu.sync_copy(rows_v, out_hbm.at[pl.ds(off, _CHUNK)])
            return carry

        lax.fori_loop(0, _NCHUNK, body, 0)

    return gather


_gather = _make_gather()


def kernel(table, indices):
    b, h = indices.shape
    idx = indices.reshape(-1).astype(jnp.int32)
    out = _gather(table, idx)
    return out.reshape(b, h, _D)


# speedup vs baseline: 1.8628x; 1.0167x over previous
import functools

import jax
import jax.numpy as jnp
from jax import lax
from jax.experimental import pallas as pl
from jax.experimental.pallas import tpu as pltpu
from jax.experimental.pallas import tpu_sc as plsc

_D = 64
_N = 16384 * 50
_NC = 2
_NS = 16
_NW = _NC * _NS
_PER_W = _N // _NW
_CHUNK = 400
_NBUF = 4
_NCHUNK = _PER_W // _CHUNK
_NSTEPS = _NCHUNK // _NBUF


def _make_gather():
    mesh = plsc.VectorSubcoreMesh(core_axis_name="c", subcore_axis_name="s")

    @functools.partial(
        pl.kernel,
        mesh=mesh,
        out_type=jax.ShapeDtypeStruct((_N, _D), jnp.float32),
        scratch_types=[
            [pltpu.VMEM((_CHUNK,), jnp.int32)] * _NBUF,
            [pltpu.VMEM((_CHUNK, _D), jnp.float32)] * _NBUF,
            [pltpu.SemaphoreType.DMA] * _NBUF,
            [pltpu.SemaphoreType.DMA] * _NBUF,
        ],
        compiler_params=pltpu.CompilerParams(use_tc_tiling_on_sc=False),
    )
    def gather(table_hbm, idx_hbm, out_hbm, idx_v, rows_v, g_sems, o_sems):
        wid = lax.axis_index("s") * _NC + lax.axis_index("c")
        base = wid * _PER_W

        def start_gather(b, chunk):
            off = base + chunk * _CHUNK
            pltpu.sync_copy(idx_hbm.at[pl.ds(off, _CHUNK)], idx_v[b])
            pltpu.async_copy(table_hbm.at[idx_v[b]], rows_v[b],
                             g_sems[b])

        def start_out(b, chunk):
            off = base + chunk * _CHUNK
            pltpu.async_copy(rows_v[b], out_hbm.at[pl.ds(off, _CHUNK)],
                             o_sems[b])

        def wait_gather(b, chunk):
            pltpu.make_async_copy(table_hbm.at[idx_v[b]], rows_v[b],
                                  g_sems[b]).wait()

        def wait_out(b, chunk):
            off = base + chunk * _CHUNK
            pltpu.make_async_copy(rows_v[b], out_hbm.at[pl.ds(off, _CHUNK)],
                                  o_sems[b]).wait()

        for b in range(_NBUF):
            start_gather(b, b)

        def step(s, carry):
            for b in range(_NBUF):
                chunk = s * _NBUF + b
                wait_gather(b, chunk)
                start_out(b, chunk)
                wait_out(b, chunk)
                start_gather(b, chunk + _NBUF)
            return carry

        lax.fori_loop(0, _NSTEPS - 1, step, 0)

        last = (_NSTEPS - 1) * _NBUF
        for b in range(_NBUF):
            wait_gather(b, last + b)
            start_out(b, last + b)
        for b in range(_NBUF):
            wait_out(b, last + b)

    return gather


_gather = _make_gather()


def kernel(table, indices):
    b, h = indices.shape
    idx = indices.reshape(-1).astype(jnp.int32)
    out = _gather(table, idx)
    return out.reshape(b, h, _D)
